# baseline (speedup 1.0000x reference)
import functools

import jax
import jax.numpy as jnp
from jax import lax
from jax.experimental import pallas as pl
from jax.experimental.pallas import tpu as pltpu
from jax.experimental.pallas import tpu_sc as plsc

N_NODES = 10000
N_EDGES = 320000
D = 128
N_LAYERS = 8

NC = 2
NS = 16
NW = NC * NS
K = 128
NB = 79
EPT = N_EDGES // NW
EPT_PAD = NB * K
TPAD = EPT_PAD - EPT
ACC_ROWS = 11264
ROWS_PER_TILE = ACC_ROWS // NS
PAD_SPAN = ACC_ROWS - N_NODES


@functools.cache
def _make_sc_scatter():
    return functools.partial(
        pl.kernel,
        mesh=plsc.VectorSubcoreMesh(core_axis_name="c", subcore_axis_name="s"),
        out_type=jax.ShapeDtypeStruct((NW, ROWS_PER_TILE, D), jnp.float32),
        scratch_types=[
            pltpu.VMEM((NB, K), jnp.int32),
            pltpu.VMEM((NB, K), jnp.int32),
            pltpu.VMEM((K, D), jnp.float32),
            pltpu.VMEM_SHARED((ACC_ROWS, D), jnp.float32),
        ],
    )(_sc_scatter_body)


def _sc_scatter_body(g_hbm, srcp_hbm, dstp_hbm, out_hbm,
                     src_v, dst_v, buf, acc_sh):
    c = lax.axis_index("c")
    s = lax.axis_index("s")
    w = c * NS + s

    zeros16 = jnp.zeros((16,), jnp.float32)

    def zbody(i, _):
        buf[i // 8, pl.ds((i % 8) * 16, 16)] = zeros16
        return _

    lax.fori_loop(0, K * 8, zbody, None)
    base = s * ROWS_PER_TILE
    for kc in range(ROWS_PER_TILE // K):
        pltpu.sync_copy(buf, acc_sh.at[pl.ds(base + kc * K, K)])
    if ROWS_PER_TILE % K:
        pltpu.sync_copy(buf, acc_sh.at[pl.ds(base + ROWS_PER_TILE - K, K)])

    pltpu.sync_copy(srcp_hbm.at[w], src_v)
    pltpu.sync_copy(dstp_hbm.at[w], dst_v)
    plsc.subcore_barrier()

    def body(j, _):
        pltpu.sync_copy(g_hbm.at[src_v.at[j]], buf)
        pltpu.sync_copy(buf, acc_sh.at[dst_v.at[j]], add=True)
        return _

    lax.fori_loop(0, NB, body, None)
    plsc.subcore_barrier()

    pltpu.sync_copy(acc_sh.at[pl.ds(base, ROWS_PER_TILE)], out_hbm.at[w])


@functools.cache
def _make_sc_degree():
    return functools.partial(
        pl.kernel,
        mesh=plsc.VectorSubcoreMesh(core_axis_name="c", subcore_axis_name="s"),
        out_type=jax.ShapeDtypeStruct((NC, NS, ROWS_PER_TILE, 16), jnp.float32),
        scratch_types=[
            pltpu.VMEM((NB, K), jnp.int32),
            pltpu.VMEM((K, 16), jnp.float32),
            pltpu.VMEM_SHARED((ACC_ROWS, 16), jnp.float32),
        ],
    )(_sc_degree_body)


def _sc_degree_body(dstp_hbm, out_hbm, dst_v, buf, deg_sh):
    c = lax.axis_index("c")
    s = lax.axis_index("s")
    w = c * NS + s

    zeros16 = jnp.zeros((16,), jnp.float32)

    def zbody(i, _):
        buf[i, :] = zeros16
        return _

    lax.fori_loop(0, K, zbody, None)
    base = s * ROWS_PER_TILE
    for kc in range(ROWS_PER_TILE // K):
        pltpu.sync_copy(buf, deg_sh.at[pl.ds(base + kc * K, K)])
    if ROWS_PER_TILE % K:
        pltpu.sync_copy(buf, deg_sh.at[pl.ds(base + ROWS_PER_TILE - K, K)])
    plsc.subcore_barrier()

    ones16 = jnp.ones((16,), jnp.float32)

    def obody(i, _):
        buf[i, :] = ones16
        return _

    lax.fori_loop(0, K, obody, None)

    pltpu.sync_copy(dstp_hbm.at[w], dst_v)

    def body(j, _):
        pltpu.sync_copy(buf, deg_sh.at[dst_v.at[j]], add=True)
        return _

    lax.fori_loop(0, NB, body, None)
    plsc.subcore_barrier()

    pltpu.sync_copy(deg_sh.at[pl.ds(base, ROWS_PER_TILE)], out_hbm.at[c, s])


_RB = 1000


def _tc_first_body(x_ref, w_ref, d0_ref, d1_ref, g_ref, dis_ref):
    dis = lax.rsqrt(d0_ref[...] + d1_ref[...] + 1.0)
    dis_ref[...] = dis
    g_ref[...] = jnp.dot(x_ref[...], w_ref[...],
                         preferred_element_type=jnp.float32) * dis


def _tc_first(x, W0, d0, d1):
    return pl.pallas_call(
        _tc_first_body,
        grid=(N_NODES // _RB,),
        in_specs=[
            pl.BlockSpec((_RB, D), lambda i: (i, 0)),
            pl.BlockSpec((D, D), lambda i: (0, 0)),
            pl.BlockSpec((_RB, 1), lambda i: (i, 0)),
            pl.BlockSpec((_RB, 1), lambda i: (i, 0)),
        ],
        out_specs=[
            pl.BlockSpec((_RB, D), lambda i: (i, 0)),
            pl.BlockSpec((_RB, 1), lambda i: (i, 0)),
        ],
        out_shape=[
            jax.ShapeDtypeStruct((N_NODES, D), jnp.float32),
            jax.ShapeDtypeStruct((N_NODES, 1), jnp.float32),
        ],
    )(x, W0, d0, d1)


def _tc_advance_body(a0_ref, a1_ref, g_ref, dis_ref, b_ref, w_ref, out_ref):
    dis = dis_ref[...]
    h = dis * (a0_ref[...] + a1_ref[...] + g_ref[...]) + b_ref[...]
    h = jnp.where(h >= 0, h, 0.1 * h)
    out_ref[...] = jnp.dot(h, w_ref[...],
                           preferred_element_type=jnp.float32) * dis


def _tc_advance(a0, a1, g, dis, b, W):
    return pl.pallas_call(
        _tc_advance_body,
        grid=(N_NODES // _RB,),
        in_specs=[
            pl.BlockSpec((_RB, D), lambda i: (i, 0)),
            pl.BlockSpec((_RB, D), lambda i: (i, 0)),
            pl.BlockSpec((_RB, D), lambda i: (i, 0)),
            pl.BlockSpec((_RB, 1), lambda i: (i, 0)),
            pl.BlockSpec((1, D), lambda i: (0, 0)),
            pl.BlockSpec((D, D), lambda i: (0, 0)),
        ],
        out_specs=pl.BlockSpec((_RB, D), lambda i: (i, 0)),
        out_shape=jax.ShapeDtypeStruct((N_NODES, D), jnp.float32),
    )(a0, a1, g, dis, b, W)


def _tc_final_body(a0_ref, a1_ref, u_ref, dis_ref, w_ref, out_ref):
    t = a0_ref[...] + a1_ref[...] + u_ref[...]
    out_ref[...] = jnp.dot(t, w_ref[...],
                           preferred_element_type=jnp.float32) * dis_ref[...]


def _tc_final(a0, a1, u, dis, Wp):
    return pl.pallas_call(
        _tc_final_body,
        grid=(N_NODES // _RB,),
        in_specs=[
            pl.BlockSpec((_RB, D), lambda i: (i, 0)),
            pl.BlockSpec((_RB, D), lambda i: (i, 0)),
            pl.BlockSpec((_RB, D), lambda i: (i, 0)),
            pl.BlockSpec((_RB, 1), lambda i: (i, 0)),
            pl.BlockSpec((D, D), lambda i: (0, 0)),
        ],
        out_specs=pl.BlockSpec((_RB, D), lambda i: (i, 0)),
        out_shape=jax.ShapeDtypeStruct((N_NODES, D), jnp.float32),
    )(a0, a1, u, dis, Wp)


def _acc_halves(a):
    a = a.reshape(NC, ACC_ROWS, D)
    return a[0, :N_NODES], a[1, :N_NODES]


def kernel(x, edge_index, Ws, bs, W_out, b_out):
    src = edge_index[0].astype(jnp.int32)
    dst = edge_index[1].astype(jnp.int32)
    wi = jnp.arange(NW, dtype=jnp.int32)[:, None]
    pi = jnp.arange(TPAD, dtype=jnp.int32)[None, :]
    pad_dst = N_NODES + ((wi % NS) * TPAD + pi) % PAD_SPAN
    srcp = jnp.concatenate(
        [src.reshape(NW, EPT), jnp.zeros((NW, TPAD), jnp.int32)], axis=1)
    dstp = jnp.concatenate([dst.reshape(NW, EPT), pad_dst], axis=1)
    srcp = srcp.reshape(NW, NB, K)
    dstp = dstp.reshape(NW, NB, K)

    scatter = _make_sc_scatter()

    degs = _make_sc_degree()(dstp).reshape(NC, ACC_ROWS, 16)
    d0 = degs[0, :N_NODES, 0:1]
    d1 = degs[1, :N_NODES, 0:1]

    g, dis = _tc_first(x, Ws[0], d0, d1)

    eye = jnp.eye(D, dtype=jnp.float32)
    for i in range(1, N_LAYERS + 1):
        a0, a1 = _acc_halves(scatter(g, srcp, dstp))
        W = Ws[i] if i < N_LAYERS else eye
        g = _tc_advance(a0, a1, g, dis, bs[i - 1][None, :], W)

    a0, a1 = _acc_halves(scatter(g, srcp, dstp))
    Wp = jnp.pad(W_out, ((0, 0), (0, D - 1)))
    o = _tc_final(a0, a1, g, dis, Wp)
    return o[:, 0] + b_out[0]


# speedup vs baseline: 1.0046x; 1.0046x over previous
import functools

import jax
import jax.numpy as jnp
from jax import lax
from jax.experimental import pallas as pl
from jax.experimental.pallas import tpu as pltpu
from jax.experimental.pallas import tpu_sc as plsc

N_NODES = 10000
N_EDGES = 320000
D = 128
N_LAYERS = 8

NC = 2
NS = 16
NW = NC * NS
K = 128
NB = 79
EPT = N_EDGES // NW
EPT_PAD = NB * K
TPAD = EPT_PAD - EPT
ACC_ROWS = 10240
ROWS_PER_TILE = ACC_ROWS // NS
PAD_SPAN = ACC_ROWS - N_NODES


@functools.cache
def _make_sc_scatter():
    return functools.partial(
        pl.kernel,
        mesh=plsc.VectorSubcoreMesh(core_axis_name="c", subcore_axis_name="s"),
        out_type=jax.ShapeDtypeStruct((NW, ROWS_PER_TILE, D), jnp.float32),
        scratch_types=[
            pltpu.VMEM((NB, K), jnp.int32),
            pltpu.VMEM((NB, K), jnp.int32),
            pltpu.VMEM((K, D), jnp.float32),
            pltpu.VMEM_SHARED((ACC_ROWS, D), jnp.float32),
        ],
    )(_sc_scatter_body)


def _sc_scatter_body(g_hbm, srcp_hbm, dstp_hbm, out_hbm,
                     src_v, dst_v, buf, acc_sh):
    c = lax.axis_index("c")
    s = lax.axis_index("s")
    w = c * NS + s

    zeros16 = jnp.zeros((16,), jnp.float32)

    def zbody(i, _):
        buf[i // 8, pl.ds((i % 8) * 16, 16)] = zeros16
        return _

    lax.fori_loop(0, K * 8, zbody, None)
    base = s * ROWS_PER_TILE
    for kc in range(ROWS_PER_TILE // K):
        pltpu.sync_copy(buf, acc_sh.at[pl.ds(base + kc * K, K)])
    if ROWS_PER_TILE % K:
        pltpu.sync_copy(buf, acc_sh.at[pl.ds(base + ROWS_PER_TILE - K, K)])

    pltpu.sync_copy(srcp_hbm.at[w], src_v)
    pltpu.sync_copy(dstp_hbm.at[w], dst_v)
    plsc.subcore_barrier()

    def body(j, _):
        pltpu.sync_copy(g_hbm.at[src_v.at[j]], buf)
        pltpu.sync_copy(buf, acc_sh.at[dst_v.at[j]], add=True)
        return _

    lax.fori_loop(0, NB, body, None)
    plsc.subcore_barrier()

    pltpu.sync_copy(acc_sh.at[pl.ds(base, ROWS_PER_TILE)], out_hbm.at[w])


@functools.cache
def _make_sc_degree():
    return functools.partial(
        pl.kernel,
        mesh=plsc.VectorSubcoreMesh(core_axis_name="c", subcore_axis_name="s"),
        out_type=jax.ShapeDtypeStruct((NC, NS, ROWS_PER_TILE, 16), jnp.float32),
        scratch_types=[
            pltpu.VMEM((NB, K), jnp.int32),
            pltpu.VMEM((K, 16), jnp.float32),
            pltpu.VMEM_SHARED((ACC_ROWS, 16), jnp.float32),
        ],
    )(_sc_degree_body)


def _sc_degree_body(dstp_hbm, out_hbm, dst_v, buf, deg_sh):
    c = lax.axis_index("c")
    s = lax.axis_index("s")
    w = c * NS + s

    zeros16 = jnp.zeros((16,), jnp.float32)

    def zbody(i, _):
        buf[i, :] = zeros16
        return _

    lax.fori_loop(0, K, zbody, None)
    base = s * ROWS_PER_TILE
    for kc in range(ROWS_PER_TILE // K):
        pltpu.sync_copy(buf, deg_sh.at[pl.ds(base + kc * K, K)])
    if ROWS_PER_TILE % K:
        pltpu.sync_copy(buf, deg_sh.at[pl.ds(base + ROWS_PER_TILE - K, K)])
    plsc.subcore_barrier()

    ones16 = jnp.ones((16,), jnp.float32)

    def obody(i, _):
        buf[i, :] = ones16
        return _

    lax.fori_loop(0, K, obody, None)

    pltpu.sync_copy(dstp_hbm.at[w], dst_v)

    def body(j, _):
        pltpu.sync_copy(buf, deg_sh.at[dst_v.at[j]], add=True)
        return _

    lax.fori_loop(0, NB, body, None)
    plsc.subcore_barrier()

    pltpu.sync_copy(deg_sh.at[pl.ds(base, ROWS_PER_TILE)], out_hbm.at[c, s])


_RB = 1000


def _tc_first_body(x_ref, w_ref, d0_ref, d1_ref, g_ref, dis_ref):
    dis = lax.rsqrt(d0_ref[...] + d1_ref[...] + 1.0)
    dis_ref[...] = dis
    g_ref[...] = jnp.dot(x_ref[...], w_ref[...],
                         preferred_element_type=jnp.float32) * dis


def _tc_first(x, W0, d0, d1):
    return pl.pallas_call(
        _tc_first_body,
        grid=(N_NODES // _RB,),
        in_specs=[
            pl.BlockSpec((_RB, D), lambda i: (i, 0)),
            pl.BlockSpec((D, D), lambda i: (0, 0)),
            pl.BlockSpec((_RB, 1), lambda i: (i, 0)),
            pl.BlockSpec((_RB, 1), lambda i: (i, 0)),
        ],
        out_specs=[
            pl.BlockSpec((_RB, D), lambda i: (i, 0)),
            pl.BlockSpec((_RB, 1), lambda i: (i, 0)),
        ],
        out_shape=[
            jax.ShapeDtypeStruct((N_NODES, D), jnp.float32),
            jax.ShapeDtypeStruct((N_NODES, 1), jnp.float32),
        ],
    )(x, W0, d0, d1)


def _tc_advance_body(a0_ref, a1_ref, g_ref, dis_ref, b_ref, w_ref, out_ref):
    dis = dis_ref[...]
    h = dis * (a0_ref[...] + a1_ref[...] + g_ref[...]) + b_ref[...]
    h = jnp.where(h >= 0, h, 0.1 * h)
    out_ref[...] = jnp.dot(h, w_ref[...],
                           preferred_element_type=jnp.float32) * dis


def _tc_advance(a0, a1, g, dis, b, W):
    return pl.pallas_call(
        _tc_advance_body,
        grid=(N_NODES // _RB,),
        in_specs=[
            pl.BlockSpec((_RB, D), lambda i: (i, 0)),
            pl.BlockSpec((_RB, D), lambda i: (i, 0)),
            pl.BlockSpec((_RB, D), lambda i: (i, 0)),
            pl.BlockSpec((_RB, 1), lambda i: (i, 0)),
            pl.BlockSpec((1, D), lambda i: (0, 0)),
            pl.BlockSpec((D, D), lambda i: (0, 0)),
        ],
        out_specs=pl.BlockSpec((_RB, D), lambda i: (i, 0)),
        out_shape=jax.ShapeDtypeStruct((N_NODES, D), jnp.float32),
    )(a0, a1, g, dis, b, W)


def _tc_final_body(a0_ref, a1_ref, u_ref, dis_ref, w_ref, out_ref):
    t = a0_ref[...] + a1_ref[...] + u_ref[...]
    out_ref[...] = jnp.dot(t, w_ref[...],
                           preferred_element_type=jnp.float32) * dis_ref[...]


def _tc_final(a0, a1, u, dis, Wp):
    return pl.pallas_call(
        _tc_final_body,
        grid=(N_NODES // _RB,),
        in_specs=[
            pl.BlockSpec((_RB, D), lambda i: (i, 0)),
            pl.BlockSpec((_RB, D), lambda i: (i, 0)),
            pl.BlockSpec((_RB, D), lambda i: (i, 0)),
            pl.BlockSpec((_RB, 1), lambda i: (i, 0)),
            pl.BlockSpec((D, D), lambda i: (0, 0)),
        ],
        out_specs=pl.BlockSpec((_RB, D), lambda i: (i, 0)),
        out_shape=jax.ShapeDtypeStruct((N_NODES, D), jnp.float32),
    )(a0, a1, u, dis, Wp)


def _acc_halves(a):
    a = a.reshape(NC, ACC_ROWS, D)
    return a[0, :N_NODES], a[1, :N_NODES]


def kernel(x, edge_index, Ws, bs, W_out, b_out):
    src = edge_index[0].astype(jnp.int32)
    dst = edge_index[1].astype(jnp.int32)
    wi = jnp.arange(NW, dtype=jnp.int32)[:, None]
    pi = jnp.arange(TPAD, dtype=jnp.int32)[None, :]
    pad_dst = N_NODES + ((wi % NS) * TPAD + pi) % PAD_SPAN
    srcp = jnp.concatenate(
        [src.reshape(NW, EPT), jnp.zeros((NW, TPAD), jnp.int32)], axis=1)
    dstp = jnp.concatenate([dst.reshape(NW, EPT), pad_dst], axis=1)
    srcp = srcp.reshape(NW, NB, K)
    dstp = dstp.reshape(NW, NB, K)

    scatter = _make_sc_scatter()

    degs = _make_sc_degree()(dstp).reshape(NC, ACC_ROWS, 16)
    d0 = degs[0, :N_NODES, 0:1]
    d1 = degs[1, :N_NODES, 0:1]

    g, dis = _tc_first(x, Ws[0], d0, d1)

    eye = jnp.eye(D, dtype=jnp.float32)
    for i in range(1, N_LAYERS + 1):
        a0, a1 = _acc_halves(scatter(g, srcp, dstp))
        W = Ws[i] if i < N_LAYERS else eye
        g = _tc_advance(a0, a1, g, dis, bs[i - 1][None, :], W)

    a0, a1 = _acc_halves(scatter(g, srcp, dstp))
    Wp = jnp.pad(W_out, ((0, 0), (0, D - 1)))
    o = _tc_final(a0, a1, g, dis, Wp)
    return o[:, 0] + b_out[0]
